# baseline (speedup 1.0000x reference)
import functools

import jax
import jax.numpy as jnp
from jax import lax
from jax.experimental import pallas as pl
from jax.experimental.pallas import tpu as pltpu
from jax.experimental.pallas import tpu_sc as plsc

N = 4096
E = 16384
BATCH = 16
L = 512
C = 64
EMB = 128
NC = 2
NS = 16
NW = NC * NS
EPW = E // NW
RPT = N // NS
ECH = 128
NCHUNK = EPW // ECH
DC = 16
NT3 = 515
MARG = 8
LBUF = 536

_sc_mesh = plsc.VectorSubcoreMesh(core_axis_name="c", subcore_axis_name="s",
                                  num_cores=NC, num_subcores=NS)


XW = 80
EPT = E // NS
TBL = E + 1


@functools.partial(
    pl.kernel,
    out_type=[jax.ShapeDtypeStruct((NC, N, XW), jnp.float32),
              jax.ShapeDtypeStruct((N, DC), jnp.float32),
              jax.ShapeDtypeStruct((NC, N, XW), jnp.float32)],
    mesh=_sc_mesh,
    scratch_types=[
        pltpu.VMEM_SHARED((N, XW), jnp.float32),
        pltpu.VMEM_SHARED((N, DC), jnp.float32),
        pltpu.VMEM((EPT // ECH, ECH), jnp.int32),
        pltpu.VMEM((ECH, DC), jnp.float32),
        pltpu.VMEM((NCHUNK, ECH), jnp.int32),
        pltpu.VMEM((NCHUNK, ECH), jnp.int32),
        pltpu.VMEM((NCHUNK, ECH, XW), jnp.float32),
        pltpu.VMEM((RPT, DC), jnp.float32),
        pltpu.VMEM((RPT, XW), jnp.float32),
        pltpu.VMEM((TBL,), jnp.float32),
        pltpu.SemaphoreType.DMA,
        pltpu.SemaphoreType.DMA,
        pltpu.SemaphoreType.DMA,
        pltpu.SemaphoreType.DMA,
        pltpu.SemaphoreType.DMA,
    ],
    compiler_params=pltpu.CompilerParams(use_tc_tiling_on_sc=False,
                                         needs_layout_passes=False),
)
def _sc_first(x_hbm, src_hbm, dst_hbm, tbl_hbm, zx_hbm, zd_hbm, ones_hbm,
              out_scat, out_deg, out_xs, acc_s, acc_d,
              dall_v, ones_v, src_v, dst_v, msg_v, degrow_v, xrow_v, tbl_v,
              g0, g1, g2, g3, ssem):
    c = lax.axis_index("c")
    s = lax.axis_index("s")
    row0 = s * RPT
    pltpu.sync_copy(zd_hbm.at[pl.ds(row0, RPT)], acc_d.at[pl.ds(row0, RPT)])
    pltpu.sync_copy(zx_hbm.at[pl.ds(row0, RPT)], acc_s.at[pl.ds(row0, RPT)])
    pltpu.sync_copy(ones_hbm, ones_v)
    pltpu.sync_copy(x_hbm.at[pl.ds(row0, RPT)], xrow_v)
    pltpu.sync_copy(tbl_hbm, tbl_v)
    for j in range(EPT // ECH):
        pltpu.sync_copy(dst_hbm.at[pl.ds(s * EPT + j * ECH, ECH)],
                        dall_v.at[j])
    base = (c * NS + s) * EPW
    for j in range(NCHUNK):
        pltpu.sync_copy(src_hbm.at[pl.ds(base + j * ECH, ECH)], src_v.at[j])
        pltpu.sync_copy(dst_hbm.at[pl.ds(base + j * ECH, ECH)], dst_v.at[j])
    plsc.subcore_barrier()
    for j in range(EPT // ECH):
        pltpu.sync_copy(ones_v, acc_d.at[dall_v.at[j]], add=True)
    plsc.subcore_barrier()
    pltpu.sync_copy(acc_d.at[pl.ds(row0, RPT)], degrow_v)

    def scale_row(r, carry):
        dv = degrow_v[r]
        di = plsc.load_gather(tbl_v, [dv.astype(jnp.int32)])
        for q in range(XW // 16):
            xrow_v[r, pl.ds(q * 16, 16)] = xrow_v[r, pl.ds(q * 16, 16)] * di
        return carry

    lax.fori_loop(0, RPT, scale_row, 0)
    pltpu.sync_copy(xrow_v, out_xs.at[c, pl.ds(row0, RPT)])

    @pl.when(c == 0)
    def _():
        pltpu.sync_copy(degrow_v, out_deg.at[pl.ds(row0, RPT)])

    plsc.subcore_barrier()
    gsems = [g0, g1, g2, g3]
    gathers = [
        pltpu.async_copy(out_xs.at[c].at[src_v.at[j]], msg_v.at[j], gsems[j])
        for j in range(NCHUNK)
    ]
    scatters = []
    for j in range(NCHUNK):
        gathers[j].wait()
        scatters.append(
            pltpu.async_copy(msg_v.at[j], acc_s.at[dst_v.at[j]], ssem,
                             add=True))
    for sc in scatters:
        sc.wait()
    plsc.subcore_barrier()
    pltpu.sync_copy(acc_s.at[pl.ds(row0, RPT)],
                    out_scat.at[c, pl.ds(row0, RPT)])


@functools.partial(
    pl.kernel,
    out_type=jax.ShapeDtypeStruct((NC, N, C), jnp.float32),
    mesh=_sc_mesh,
    scratch_types=[
        pltpu.VMEM_SHARED((N, C), jnp.float32),
        pltpu.VMEM((NCHUNK, ECH), jnp.int32),
        pltpu.VMEM((NCHUNK, ECH), jnp.int32),
        pltpu.VMEM((NCHUNK, ECH, C), jnp.float32),
        pltpu.SemaphoreType.DMA,
        pltpu.SemaphoreType.DMA,
        pltpu.SemaphoreType.DMA,
        pltpu.SemaphoreType.DMA,
        pltpu.SemaphoreType.DMA,
    ],
    compiler_params=pltpu.CompilerParams(use_tc_tiling_on_sc=False),
)
def _gcn_scatter(hs_hbm, src_hbm, dst_hbm, zeros_hbm, out_hbm,
                 acc, src_v, dst_v, msg_v, g0, g1, g2, g3, ssem):
    c = lax.axis_index("c")
    s = lax.axis_index("s")
    row0 = s * RPT
    pltpu.sync_copy(zeros_hbm.at[pl.ds(row0, RPT)], acc.at[pl.ds(row0, RPT)])
    base = (c * NS + s) * EPW
    for j in range(NCHUNK):
        pltpu.sync_copy(src_hbm.at[pl.ds(base + j * ECH, ECH)], src_v.at[j])
        pltpu.sync_copy(dst_hbm.at[pl.ds(base + j * ECH, ECH)], dst_v.at[j])
    plsc.subcore_barrier()
    gsems = [g0, g1, g2, g3]
    gathers = [
        pltpu.async_copy(hs_hbm.at[src_v.at[j]], msg_v.at[j], gsems[j])
        for j in range(NCHUNK)
    ]
    scatters = []
    for j in range(NCHUNK):
        gathers[j].wait()
        scatters.append(
            pltpu.async_copy(msg_v.at[j], acc.at[dst_v.at[j]], ssem, add=True))
    for sc in scatters:
        sc.wait()
    plsc.subcore_barrier()
    pltpu.sync_copy(acc.at[pl.ds(row0, RPT)], out_hbm.at[c, pl.ds(row0, RPT)])


def _dinv_from(deg):
    return lax.rsqrt(jnp.maximum(deg[:, 0] + 1.0, 1.0))


def _tc_first_body(scat_ref, xs_ref, w_ref, b_ref, deg_ref, out_ref):
    dinv = _dinv_from(deg_ref[...])
    sc = scat_ref[...]
    agg = dinv[:, None] * (sc[0] + sc[1] + xs_ref[...][0])
    h = jnp.maximum(jnp.dot(agg, w_ref[...],
                            preferred_element_type=jnp.float32)
                    + b_ref[...], 0.0)
    out_ref[...] = h * dinv[:, None]


_tc_first = pl.pallas_call(
    _tc_first_body,
    out_shape=jax.ShapeDtypeStruct((N, C), jnp.float32),
)


def _tc_mid_body(scat_ref, hs_ref, b_ref, w_ref, deg_ref, out_ref):
    dinv = _dinv_from(deg_ref[...])
    sc = scat_ref[...]
    agg = dinv[:, None] * (sc[0] + sc[1] + hs_ref[...])
    h = jnp.maximum(jnp.dot(agg, w_ref[...],
                            preferred_element_type=jnp.float32)
                    + b_ref[...], 0.0)
    out_ref[...] = h * dinv[:, None]


_tc_mid = pl.pallas_call(
    _tc_mid_body,
    out_shape=jax.ShapeDtypeStruct((N, C), jnp.float32),
)


def _roll0(x, s):
    n = x.shape[0]
    s = s % n
    if s == 0:
        return x
    return jnp.concatenate([x[n - s:], x[:n - s]], axis=0)


LSTK = BATCH * LBUF


def _tc_big_body(tok_ref, emb_ref, w1_ref, c1_ref, w2_ref, c2_ref,
                 w3_ref, c3_ref, scat_ref, hs_ref, wg3_ref, b3_ref, deg_ref,
                 bidx_ref, fw1_ref, fb1_ref, fw2_ref, fb2_ref, fw3_ref,
                 fb3_ref, out_ref):
    tok = tok_ref[...]
    oh = (tok[:, :, None] == lax.broadcasted_iota(jnp.int32, (BATCH, L, 32), 2))
    x0 = jnp.dot(oh.astype(jnp.float32).reshape(BATCH * L, 32), emb_ref[...],
                 preferred_element_type=jnp.float32)
    stk = jnp.concatenate(
        [jnp.zeros((BATCH, MARG, EMB), jnp.float32),
         x0.reshape(BATCH, L, EMB),
         jnp.zeros((BATCH, LBUF - MARG - L, EMB), jnp.float32)],
        axis=1).reshape(LSTK, EMB)
    tm = lax.rem(lax.broadcasted_iota(jnp.int32, (LSTK, 1), 0), LBUF)

    def layer(buf_in, w_ref, b_row, k, p, lout):
        w = w_ref[...]
        xs = [buf_in]
        for _ in range(3):
            xs.append(_roll0(xs[-1], -1))
        y = jnp.zeros((LSTK, C), jnp.float32)
        for q in range(k // 4):
            u = jnp.zeros((LSTK, C), jnp.float32)
            for r in range(4):
                u = u + jnp.dot(xs[r], w[4 * q + r],
                                preferred_element_type=jnp.float32)
            y = y + _roll0(u, p - 4 * q)
        y = jnp.maximum(y + b_row, 0.0)
        mask = (tm >= MARG) & (tm < MARG + lout)
        return jnp.where(mask, y, 0.0)

    y1 = layer(stk, w1_ref, c1_ref[...], 4, 2, L + 1)
    y2 = layer(y1, w2_ref, c2_ref[...], 8, 4, L + 2)
    y3 = layer(y2, w3_ref, c3_ref[...], 12, 6, L + 3)
    tctx = jnp.sum(y3.reshape(BATCH, LBUF, C), axis=1) / float(NT3)

    dinv = _dinv_from(deg_ref[...])
    sc = scat_ref[...]
    agg = dinv[:, None] * (sc[0] + sc[1] + hs_ref[...])
    feats = jnp.maximum(
        jnp.dot(agg, wg3_ref[...], preferred_element_type=jnp.float32)
        + b3_ref[...], 0.0)
    ohb = (bidx_ref[...] == lax.broadcasted_iota(jnp.int32, (BATCH, N), 0))
    drug_ctx = jnp.dot(ohb.astype(jnp.float32), feats,
                       preferred_element_type=jnp.float32) / float(NT3)
    inter = jnp.concatenate([drug_ctx, tctx], axis=1)
    z = jnp.maximum(jnp.dot(inter, fw1_ref[...],
                            preferred_element_type=jnp.float32)
                    + fb1_ref[...], 0.0)
    z = jnp.maximum(jnp.dot(z, fw2_ref[...],
                            preferred_element_type=jnp.float32)
                    + fb2_ref[...], 0.0)
    o = jnp.sum(z * fw3_ref[...], axis=1) + fb3_ref[0, 0]
    out_ref[...] = o[None, :]


_tc_big = pl.pallas_call(
    _tc_big_body,
    out_shape=jax.ShapeDtypeStruct((1, BATCH), jnp.float32),
)


def kernel(drug_x, edge_index, batch_idx, target_tokens, embed,
           W1, b1, W2, b2, W3, b3,
           cw1, cb1, cw2, cb2, cw3, cb3,
           W_a, fw1, fb1, fw2, fb2, fw3, fb3):
    src = edge_index[0].astype(jnp.int32)
    dst = edge_index[1].astype(jnp.int32)
    x80 = jnp.zeros((N, XW), jnp.float32).at[:, :78].set(drug_x)
    w1p = jnp.zeros((XW, C), jnp.float32).at[:78].set(W1)
    tbl = lax.rsqrt(jnp.arange(1, TBL + 1, dtype=jnp.float32))
    zeros_x = jnp.zeros((N, XW), jnp.float32)
    zeros_c = jnp.zeros((N, C), jnp.float32)
    zeros_d = jnp.zeros((N, DC), jnp.float32)
    ones_d = jnp.ones((ECH, DC), jnp.float32)

    scatx, degf, xsf = _sc_first(x80, src, dst, tbl, zeros_x, zeros_d, ones_d)
    hs1 = _tc_first(scatx, xsf, w1p, b1.reshape(1, C), degf)
    scat2 = _gcn_scatter(hs1, src, dst, zeros_c)
    hs2 = _tc_mid(scat2, hs1, b2.reshape(1, C), W2, degf)
    scat3 = _gcn_scatter(hs2, src, dst, zeros_c)

    emb_pad = jnp.zeros((32, EMB), jnp.float32).at[:26].set(embed)
    out = _tc_big(
        target_tokens.astype(jnp.int32), emb_pad,
        jnp.transpose(cw1, (2, 1, 0)), cb1.reshape(1, C),
        jnp.transpose(cw2, (2, 1, 0)), cb2.reshape(1, C),
        jnp.transpose(cw3, (2, 1, 0)), cb3.reshape(1, C),
        scat3, hs2, W3, b3.reshape(1, C), degf,
        batch_idx.astype(jnp.int32).reshape(1, N),
        fw1, fb1.reshape(1, 256), fw2, fb2.reshape(1, 128),
        fw3.reshape(1, 128), fb3.reshape(1, 1),
    )
    return out.reshape(BATCH)


# speedup vs baseline: 1.1221x; 1.1221x over previous
import functools

import jax
import jax.numpy as jnp
from jax import lax
from jax.experimental import pallas as pl
from jax.experimental.pallas import tpu as pltpu
from jax.experimental.pallas import tpu_sc as plsc

N = 4096
E = 16384
BATCH = 16
L = 512
C = 64
EMB = 128
NC = 2
NS = 16
NW = NC * NS
EPW = E // NW
RPT = N // NS
ECH = 128
NCHUNK = EPW // ECH
DC = 16
NT3 = 515
MARG = 8
LBUF = 536

_sc_mesh = plsc.VectorSubcoreMesh(core_axis_name="c", subcore_axis_name="s",
                                  num_cores=NC, num_subcores=NS)


@functools.partial(
    pl.kernel,
    out_type=jax.ShapeDtypeStruct((NC, N, DC), jnp.float32),
    mesh=_sc_mesh,
    scratch_types=[
        pltpu.VMEM_SHARED((N, DC), jnp.float32),
        pltpu.VMEM((NCHUNK, ECH), jnp.int32),
        pltpu.VMEM((ECH, DC), jnp.float32),
        pltpu.VMEM((RPT, DC), jnp.float32),
    ],
    compiler_params=pltpu.CompilerParams(use_tc_tiling_on_sc=False,
                                         needs_layout_passes=False),
)
def _deg_scatter(dst_hbm, out_hbm, acc, dst_v, ones_v, zb_v):
    c = lax.axis_index("c")
    s = lax.axis_index("s")
    row0 = s * RPT

    def fill(r, carry):
        ones_v[r] = jnp.full((DC,), 1.0, jnp.float32)
        zb_v[r] = jnp.zeros((DC,), jnp.float32)
        zb_v[r + ECH] = jnp.zeros((DC,), jnp.float32)
        return carry

    lax.fori_loop(0, ECH, fill, 0)
    pltpu.sync_copy(zb_v, acc.at[pl.ds(row0, RPT)])
    base = (c * NS + s) * EPW
    for j in range(NCHUNK):
        pltpu.sync_copy(dst_hbm.at[pl.ds(base + j * ECH, ECH)], dst_v.at[j])
    plsc.subcore_barrier()
    for j in range(NCHUNK):
        pltpu.sync_copy(ones_v, acc.at[dst_v.at[j]], add=True)
    plsc.subcore_barrier()
    pltpu.sync_copy(acc.at[pl.ds(row0, RPT)], out_hbm.at[c, pl.ds(row0, RPT)])


@functools.partial(
    pl.kernel,
    out_type=jax.ShapeDtypeStruct((NC, N, C), jnp.float32),
    mesh=_sc_mesh,
    scratch_types=[
        pltpu.VMEM_SHARED((N, C), jnp.float32),
        pltpu.VMEM((NCHUNK, ECH), jnp.int32),
        pltpu.VMEM((NCHUNK, ECH), jnp.int32),
        pltpu.VMEM((NCHUNK, ECH, C), jnp.float32),
        pltpu.VMEM((RPT, C), jnp.float32),
        pltpu.SemaphoreType.DMA,
        pltpu.SemaphoreType.DMA,
        pltpu.SemaphoreType.DMA,
        pltpu.SemaphoreType.DMA,
        pltpu.SemaphoreType.DMA,
    ],
    compiler_params=pltpu.CompilerParams(use_tc_tiling_on_sc=False,
                                         needs_layout_passes=False),
)
def _gcn_scatter(hs_hbm, src_hbm, dst_hbm, out_hbm,
                 acc, src_v, dst_v, msg_v, zb_v, g0, g1, g2, g3, ssem):
    c = lax.axis_index("c")
    s = lax.axis_index("s")
    row0 = s * RPT

    def fill(r, carry):
        for q in range(C // 16):
            zb_v[r, pl.ds(q * 16, 16)] = jnp.zeros((16,), jnp.float32)
        return carry

    lax.fori_loop(0, RPT, fill, 0)
    pltpu.sync_copy(zb_v, acc.at[pl.ds(row0, RPT)])
    base = (c * NS + s) * EPW
    for j in range(NCHUNK):
        pltpu.sync_copy(src_hbm.at[pl.ds(base + j * ECH, ECH)], src_v.at[j])
        pltpu.sync_copy(dst_hbm.at[pl.ds(base + j * ECH, ECH)], dst_v.at[j])
    plsc.subcore_barrier()
    gsems = [g0, g1, g2, g3]
    gathers = [
        pltpu.async_copy(hs_hbm.at[src_v.at[j]], msg_v.at[j], gsems[j])
        for j in range(NCHUNK)
    ]
    scatters = []
    for j in range(NCHUNK):
        gathers[j].wait()
        scatters.append(
            pltpu.async_copy(msg_v.at[j], acc.at[dst_v.at[j]], ssem, add=True))
    for sc in scatters:
        sc.wait()
    plsc.subcore_barrier()
    pltpu.sync_copy(acc.at[pl.ds(row0, RPT)], out_hbm.at[c, pl.ds(row0, RPT)])


def _dinv_from(degp):
    deg = degp[0, :, 0] + degp[1, :, 0] + 1.0
    return lax.rsqrt(jnp.maximum(deg, 1.0))


def _tc_first_body(x_ref, w_ref, degp_ref, out_ref):
    h = jnp.dot(x_ref[...], w_ref[...], preferred_element_type=jnp.float32)
    dinv = _dinv_from(degp_ref[...])
    out_ref[...] = h * dinv[:, None]


_tc_first = pl.pallas_call(
    _tc_first_body,
    out_shape=jax.ShapeDtypeStruct((N, C), jnp.float32),
)


def _tc_mid_body(scat_ref, hs_ref, b_ref, w_ref, degp_ref, out_ref):
    dinv = _dinv_from(degp_ref[...])
    sc = scat_ref[...]
    pre = dinv[:, None] * (sc[0] + sc[1] + hs_ref[...]) + b_ref[...]
    h = jnp.maximum(pre, 0.0)
    out_ref[...] = jnp.dot(h, w_ref[...],
                           preferred_element_type=jnp.float32) * dinv[:, None]


_tc_mid = pl.pallas_call(
    _tc_mid_body,
    out_shape=jax.ShapeDtypeStruct((N, C), jnp.float32),
)


LSTK = BATCH * LBUF


def _roll0(x, s):
    n = x.shape[0]
    s = s % n
    if s == 0:
        return x
    return jnp.concatenate([x[n - s:], x[:n - s]], axis=0)


def _tc_big_body(tok_ref, emb_ref, w1_ref, c1_ref, w2_ref, c2_ref,
                 w3_ref, c3_ref, scat_ref, hs_ref, b3_ref, degp_ref,
                 bidx_ref, fw1_ref, fb1_ref, fw2_ref, fb2_ref, fw3_ref,
                 fb3_ref, out_ref):
    tok = tok_ref[...]
    oh = (tok[:, :, None] == lax.broadcasted_iota(jnp.int32, (BATCH, L, 32), 2))
    x0 = jnp.dot(oh.astype(jnp.float32).reshape(BATCH * L, 32), emb_ref[...],
                 preferred_element_type=jnp.float32)
    stk = jnp.concatenate(
        [jnp.zeros((BATCH, MARG, EMB), jnp.float32),
         x0.reshape(BATCH, L, EMB),
         jnp.zeros((BATCH, LBUF - MARG - L, EMB), jnp.float32)],
        axis=1).reshape(LSTK, EMB)
    tm = lax.rem(lax.broadcasted_iota(jnp.int32, (LSTK, 1), 0), LBUF)

    def layer(buf_in, w_ref, b_row, k, p, lout):
        w = w_ref[...]
        xs = [buf_in]
        for _ in range(3):
            xs.append(_roll0(xs[-1], -1))
        y = jnp.zeros((LSTK, C), jnp.float32)
        for q in range(k // 4):
            u = jnp.zeros((LSTK, C), jnp.float32)
            for r in range(4):
                u = u + jnp.dot(xs[r], w[4 * q + r],
                                preferred_element_type=jnp.float32)
            y = y + _roll0(u, p - 4 * q)
        y = jnp.maximum(y + b_row, 0.0)
        mask = (tm >= MARG) & (tm < MARG + lout)
        return jnp.where(mask, y, 0.0)

    y1 = layer(stk, w1_ref, c1_ref[...], 4, 2, L + 1)
    y2 = layer(y1, w2_ref, c2_ref[...], 8, 4, L + 2)
    y3 = layer(y2, w3_ref, c3_ref[...], 12, 6, L + 3)
    tctx = jnp.sum(y3.reshape(BATCH, LBUF, C), axis=1) / float(NT3)

    dinv = _dinv_from(degp_ref[...])
    sc = scat_ref[...]
    feats = jnp.maximum(
        dinv[:, None] * (sc[0] + sc[1] + hs_ref[...]) + b3_ref[...], 0.0)
    ohb = (bidx_ref[...] == lax.broadcasted_iota(jnp.int32, (BATCH, N), 0))
    drug_ctx = jnp.dot(ohb.astype(jnp.float32), feats,
                       preferred_element_type=jnp.float32) / float(NT3)
    inter = jnp.concatenate([drug_ctx, tctx], axis=1)
    z = jnp.maximum(jnp.dot(inter, fw1_ref[...],
                            preferred_element_type=jnp.float32)
                    + fb1_ref[...], 0.0)
    z = jnp.maximum(jnp.dot(z, fw2_ref[...],
                            preferred_element_type=jnp.float32)
                    + fb2_ref[...], 0.0)
    o = jnp.sum(z * fw3_ref[...], axis=1) + fb3_ref[0, 0]
    out_ref[...] = o[None, :]


_tc_big = pl.pallas_call(
    _tc_big_body,
    out_shape=jax.ShapeDtypeStruct((1, BATCH), jnp.float32),
)


def kernel(drug_x, edge_index, batch_idx, target_tokens, embed,
           W1, b1, W2, b2, W3, b3,
           cw1, cb1, cw2, cb2, cw3, cb3,
           W_a, fw1, fb1, fw2, fb2, fw3, fb3):
    src = edge_index[0].astype(jnp.int32)
    dst = edge_index[1].astype(jnp.int32)

    degp = _deg_scatter(dst)
    h0s = _tc_first(drug_x, W1, degp)
    scat1 = _gcn_scatter(h0s, src, dst)
    h1s = _tc_mid(scat1, h0s, b1.reshape(1, C), W2, degp)
    scat2 = _gcn_scatter(h1s, src, dst)
    h2s = _tc_mid(scat2, h1s, b2.reshape(1, C), W3, degp)
    scat3 = _gcn_scatter(h2s, src, dst)

    emb_pad = jnp.zeros((32, EMB), jnp.float32).at[:26].set(embed)
    out = _tc_big(
        target_tokens.astype(jnp.int32), emb_pad,
        jnp.transpose(cw1, (2, 1, 0)), cb1.reshape(1, C),
        jnp.transpose(cw2, (2, 1, 0)), cb2.reshape(1, C),
        jnp.transpose(cw3, (2, 1, 0)), cb3.reshape(1, C),
        scat3, h2s, b3.reshape(1, C), degp,
        batch_idx.astype(jnp.int32).reshape(1, N),
        fw1, fb1.reshape(1, 256), fw2, fb2.reshape(1, 128),
        fw3.reshape(1, 128), fb3.reshape(1, 1),
    )
    return out.reshape(BATCH)


# speedup vs baseline: 1.1758x; 1.0478x over previous
import functools

import jax
import jax.numpy as jnp
from jax import lax
from jax.experimental import pallas as pl
from jax.experimental.pallas import tpu as pltpu
from jax.experimental.pallas import tpu_sc as plsc

N = 4096
E = 16384
BATCH = 16
L = 512
C = 64
EMB = 128
NC = 2
NS = 16
NW = NC * NS
EPW = E // NW
RPT = N // NS
ECH = 128
NCHUNK = EPW // ECH
DC = 16
NT3 = 515
MARG = 8
LBUF = 536

_sc_mesh = plsc.VectorSubcoreMesh(core_axis_name="c", subcore_axis_name="s",
                                  num_cores=NC, num_subcores=NS)


@functools.partial(
    pl.kernel,
    out_type=jax.ShapeDtypeStruct((NC, N, DC), jnp.float32),
    mesh=_sc_mesh,
    scratch_types=[
        pltpu.VMEM_SHARED((N, DC), jnp.float32),
        pltpu.VMEM((NCHUNK, ECH), jnp.int32),
        pltpu.VMEM((ECH, DC), jnp.float32),
        pltpu.VMEM((RPT, DC), jnp.float32),
    ],
    compiler_params=pltpu.CompilerParams(use_tc_tiling_on_sc=False,
                                         needs_layout_passes=False),
)
def _deg_scatter(dst_hbm, out_hbm, acc, dst_v, ones_v, zb_v):
    c = lax.axis_index("c")
    s = lax.axis_index("s")
    row0 = s * RPT

    def fill(r, carry):
        ones_v[r] = jnp.full((DC,), 1.0, jnp.float32)
        zb_v[r] = jnp.zeros((DC,), jnp.float32)
        zb_v[r + ECH] = jnp.zeros((DC,), jnp.float32)
        return carry

    lax.fori_loop(0, ECH, fill, 0)
    pltpu.sync_copy(zb_v, acc.at[pl.ds(row0, RPT)])
    base = (c * NS + s) * EPW
    for j in range(NCHUNK):
        pltpu.sync_copy(dst_hbm.at[pl.ds(base + j * ECH, ECH)], dst_v.at[j])
    plsc.subcore_barrier()
    for j in range(NCHUNK):
        pltpu.sync_copy(ones_v, acc.at[dst_v.at[j]], add=True)
    plsc.subcore_barrier()
    pltpu.sync_copy(acc.at[pl.ds(row0, RPT)], out_hbm.at[c, pl.ds(row0, RPT)])


@functools.partial(
    pl.kernel,
    out_type=jax.ShapeDtypeStruct((NC, N, C), jnp.float32),
    mesh=_sc_mesh,
    scratch_types=[
        pltpu.VMEM_SHARED((N, C), jnp.float32),
        pltpu.VMEM((NCHUNK, ECH), jnp.int32),
        pltpu.VMEM((NCHUNK, ECH), jnp.int32),
        pltpu.VMEM((NCHUNK, ECH, C), jnp.float32),
        pltpu.VMEM((RPT, C), jnp.float32),
        pltpu.SemaphoreType.DMA,
        pltpu.SemaphoreType.DMA,
        pltpu.SemaphoreType.DMA,
        pltpu.SemaphoreType.DMA,
        pltpu.SemaphoreType.DMA,
    ],
    compiler_params=pltpu.CompilerParams(use_tc_tiling_on_sc=False,
                                         needs_layout_passes=False),
)
def _gcn_scatter(hs_hbm, src_hbm, dst_hbm, out_hbm,
                 acc, src_v, dst_v, msg_v, zb_v, g0, g1, g2, g3, ssem):
    c = lax.axis_index("c")
    s = lax.axis_index("s")
    row0 = s * RPT

    def fill(r, carry):
        for q in range(C // 16):
            zb_v[r, pl.ds(q * 16, 16)] = jnp.zeros((16,), jnp.float32)
        return carry

    lax.fori_loop(0, RPT, fill, 0)
    pltpu.sync_copy(zb_v, acc.at[pl.ds(row0, RPT)])
    base = (c * NS + s) * EPW
    for j in range(NCHUNK):
        pltpu.sync_copy(src_hbm.at[pl.ds(base + j * ECH, ECH)], src_v.at[j])
        pltpu.sync_copy(dst_hbm.at[pl.ds(base + j * ECH, ECH)], dst_v.at[j])
    plsc.subcore_barrier()
    gsems = [g0, g1, g2, g3]
    gathers = [
        pltpu.async_copy(hs_hbm.at[src_v.at[j]], msg_v.at[j], gsems[j])
        for j in range(NCHUNK)
    ]
    scatters = []
    for j in range(NCHUNK):
        gathers[j].wait()
        scatters.append(
            pltpu.async_copy(msg_v.at[j], acc.at[dst_v.at[j]], ssem, add=True))
    for sc in scatters:
        sc.wait()
    plsc.subcore_barrier()
    pltpu.sync_copy(acc.at[pl.ds(row0, RPT)], out_hbm.at[c, pl.ds(row0, RPT)])


def _dinv_from(degp):
    deg = degp[0, :, 0] + degp[1, :, 0] + 1.0
    return lax.rsqrt(jnp.maximum(deg, 1.0))


def _tc_first_body(x_ref, w_ref, degp_ref, out_ref):
    h = jnp.dot(x_ref[...], w_ref[...], preferred_element_type=jnp.float32)
    dinv = _dinv_from(degp_ref[...])
    out_ref[...] = h * dinv[:, None]


_tc_first = pl.pallas_call(
    _tc_first_body,
    out_shape=jax.ShapeDtypeStruct((N, C), jnp.float32),
)


def _tc_mid_body(scat_ref, hs_ref, b_ref, w_ref, degp_ref, out_ref):
    dinv = _dinv_from(degp_ref[...])
    sc = scat_ref[...]
    pre = dinv[:, None] * (sc[0] + sc[1] + hs_ref[...]) + b_ref[...]
    h = jnp.maximum(pre, 0.0)
    out_ref[...] = jnp.dot(h, w_ref[...],
                           preferred_element_type=jnp.float32) * dinv[:, None]


_tc_mid = pl.pallas_call(
    _tc_mid_body,
    out_shape=jax.ShapeDtypeStruct((N, C), jnp.float32),
)


LSTK = BATCH * LBUF


def _roll0(x, s):
    n = x.shape[0]
    s = s % n
    if s == 0:
        return x
    return jnp.concatenate([x[n - s:], x[:n - s]], axis=0)


def _tc_cnn_body(tok_ref, emb_ref, w1_ref, c1_ref, w2_ref, c2_ref,
                 w3_ref, c3_ref, out_ref):
    tok = tok_ref[...]
    oh = (tok[:, :, None] == lax.broadcasted_iota(jnp.int32, (BATCH, L, 32), 2))
    x0 = jnp.dot(oh.astype(jnp.float32).reshape(BATCH * L, 32), emb_ref[...],
                 preferred_element_type=jnp.float32)
    stk = jnp.concatenate(
        [jnp.zeros((BATCH, MARG, EMB), jnp.float32),
         x0.reshape(BATCH, L, EMB),
         jnp.zeros((BATCH, LBUF - MARG - L, EMB), jnp.float32)],
        axis=1).reshape(LSTK, EMB)
    tm = lax.rem(lax.broadcasted_iota(jnp.int32, (LSTK, 1), 0), LBUF)

    def layer(buf_in, w_ref, b_row, k, p, lout):
        w = w_ref[...]
        xs = [buf_in]
        for _ in range(3):
            xs.append(_roll0(xs[-1], -1))
        y = jnp.zeros((LSTK, C), jnp.float32)
        for q in range(k // 4):
            u = jnp.zeros((LSTK, C), jnp.float32)
            for r in range(4):
                u = u + jnp.dot(xs[r], w[4 * q + r],
                                preferred_element_type=jnp.float32)
            y = y + _roll0(u, p - 4 * q)
        y = jnp.maximum(y + b_row, 0.0)
        mask = (tm >= MARG) & (tm < MARG + lout)
        return jnp.where(mask, y, 0.0)

    y1 = layer(stk, w1_ref, c1_ref[...], 4, 2, L + 1)
    y2 = layer(y1, w2_ref, c2_ref[...], 8, 4, L + 2)
    y3 = layer(y2, w3_ref, c3_ref[...], 12, 6, L + 3)
    out_ref[...] = jnp.sum(y3.reshape(BATCH, LBUF, C), axis=1) / float(NT3)


_tc_cnn = pl.pallas_call(
    _tc_cnn_body,
    out_shape=jax.ShapeDtypeStruct((BATCH, C), jnp.float32),
)


def _tc_final_body(scat_ref, hs_ref, b3_ref, degp_ref, bidx_ref, tctx_ref,
                   fw1_ref, fb1_ref, fw2_ref, fb2_ref, fw3_ref, fb3_ref,
                   out_ref):
    tctx = tctx_ref[...]
    dinv = _dinv_from(degp_ref[...])
    sc = scat_ref[...]
    feats = jnp.maximum(
        dinv[:, None] * (sc[0] + sc[1] + hs_ref[...]) + b3_ref[...], 0.0)
    ohb = (bidx_ref[...] == lax.broadcasted_iota(jnp.int32, (BATCH, N), 0))
    drug_ctx = jnp.dot(ohb.astype(jnp.float32), feats,
                       preferred_element_type=jnp.float32) / float(NT3)
    inter = jnp.concatenate([drug_ctx, tctx], axis=1)
    z = jnp.maximum(jnp.dot(inter, fw1_ref[...],
                            preferred_element_type=jnp.float32)
                    + fb1_ref[...], 0.0)
    z = jnp.maximum(jnp.dot(z, fw2_ref[...],
                            preferred_element_type=jnp.float32)
                    + fb2_ref[...], 0.0)
    o = jnp.sum(z * fw3_ref[...], axis=1) + fb3_ref[0, 0]
    out_ref[...] = o[None, :]


_tc_final = pl.pallas_call(
    _tc_final_body,
    out_shape=jax.ShapeDtypeStruct((1, BATCH), jnp.float32),
)


def kernel(drug_x, edge_index, batch_idx, target_tokens, embed,
           W1, b1, W2, b2, W3, b3,
           cw1, cb1, cw2, cb2, cw3, cb3,
           W_a, fw1, fb1, fw2, fb2, fw3, fb3):
    src = edge_index[0].astype(jnp.int32)
    dst = edge_index[1].astype(jnp.int32)

    emb_pad = jnp.zeros((32, EMB), jnp.float32).at[:26].set(embed)
    tctx = _tc_cnn(
        target_tokens.astype(jnp.int32), emb_pad,
        jnp.transpose(cw1, (2, 1, 0)), cb1.reshape(1, C),
        jnp.transpose(cw2, (2, 1, 0)), cb2.reshape(1, C),
        jnp.transpose(cw3, (2, 1, 0)), cb3.reshape(1, C),
    )

    degp = _deg_scatter(dst)
    h0s = _tc_first(drug_x, W1, degp)
    scat1 = _gcn_scatter(h0s, src, dst)
    h1s = _tc_mid(scat1, h0s, b1.reshape(1, C), W2, degp)
    scat2 = _gcn_scatter(h1s, src, dst)
    h2s = _tc_mid(scat2, h1s, b2.reshape(1, C), W3, degp)
    scat3 = _gcn_scatter(h2s, src, dst)

    out = _tc_final(
        scat3, h2s, b3.reshape(1, C), degp,
        batch_idx.astype(jnp.int32).reshape(1, N), tctx,
        fw1, fb1.reshape(1, 256), fw2, fb2.reshape(1, 128),
        fw3.reshape(1, 128), fb3.reshape(1, 1),
    )
    return out.reshape(BATCH)
